# baseline (speedup 1.0000x reference)
import jax
import jax.numpy as jnp
from jax.experimental import pallas as pl

N, T, P, D = 8, 48, 325, 512
M = N * T * P

_BM = 1600
_GRID = M // _BM


def _body(x_ref, mon_ref, tw_ref, tb_ref, vw_ref, vb_ref, et_ref, ut_ref, out_ref):
    value = x_ref[:, 0:1]
    time = x_ref[:, 1:2]
    mon = mon_ref[:, 0:1] != 0
    invalid = jnp.isnan(value)

    tw = tw_ref[0:1, :]
    tb = tb_ref[0:1, :]
    vw = vw_ref[0:1, :]
    vb = vb_ref[0:1, :]
    et = et_ref[0:1, :]
    ut = ut_ref[0:1, :]

    bias = jnp.where(mon, jnp.where(invalid, tb + et, tb + vb), tb + ut)
    coef = jnp.where(mon & ~invalid, jnp.where(invalid, 0.0, value), 0.0)
    out_ref[...] = time * tw + coef * vw + bias


def kernel(x, monitor_mask, time_emb_w, time_emb_b, value_emb_w, value_emb_b,
           empty_token, unmonitored_token):
    x2 = x.reshape(M, 2)
    mon = monitor_mask.reshape(M, 1).astype(jnp.int32)
    vec = lambda v: v.reshape(1, D)
    full = pl.BlockSpec((1, D), lambda i: (0, 0))
    out = pl.pallas_call(
        _body,
        grid=(_GRID,),
        in_specs=[
            pl.BlockSpec((_BM, 2), lambda i: (i, 0)),
            pl.BlockSpec((_BM, 1), lambda i: (i, 0)),
            full, full, full, full, full, full,
        ],
        out_specs=pl.BlockSpec((_BM, D), lambda i: (i, 0)),
        out_shape=jax.ShapeDtypeStruct((M, D), jnp.float32),
    )(x2, mon, vec(time_emb_w), vec(time_emb_b), vec(value_emb_w),
      vec(value_emb_b), vec(empty_token), vec(unmonitored_token))
    return out.reshape(N, T, P, D)


# speedup vs baseline: 1.0131x; 1.0131x over previous
import jax
import jax.numpy as jnp
from jax.experimental import pallas as pl

_N, _T, _P, _D = 8, 48, 325, 512
_M = _N * _T * _P

_BM = 1920
_GRID = _M // _BM


def _body(x_ref, mon_ref, tw_ref, tb_ref, vw_ref, vb_ref, et_ref, ut_ref,
          out_ref):
    v = x_ref[:, 0:1]
    t = x_ref[:, 1:2]
    mon = mon_ref[:, 0:1] > 0.5
    inv = jnp.isnan(v)
    coef = jnp.where(mon & ~inv, v, 0.0)
    m_emp = jnp.where(mon & inv, 1.0, 0.0)
    m_un = jnp.where(mon, 0.0, 1.0)

    tw = tw_ref[0:1, :]
    tb = tb_ref[0:1, :]
    vw = vw_ref[0:1, :]
    vb = vb_ref[0:1, :]
    et = et_ref[0:1, :]
    ut = ut_ref[0:1, :]

    out_ref[...] = (t * tw + coef * vw + (tb + vb)
                    + m_emp * (et - vb) + m_un * (ut - vb))


def kernel(x, monitor_mask, time_emb_w, time_emb_b, value_emb_w, value_emb_b,
           empty_token, unmonitored_token):
    xm = x.reshape(_M, 2)
    mon = monitor_mask.reshape(_M, 1).astype(jnp.float32)
    vec = lambda v: v.reshape(1, _D)
    full = pl.BlockSpec((1, _D), lambda i: (0, 0))
    col = pl.BlockSpec((_BM, 2), lambda i: (i, 0))
    colm = pl.BlockSpec((_BM, 1), lambda i: (i, 0))
    out = pl.pallas_call(
        _body,
        grid=(_GRID,),
        in_specs=[col, colm, full, full, full, full, full, full],
        out_specs=pl.BlockSpec((_BM, _D), lambda i: (i, 0)),
        out_shape=jax.ShapeDtypeStruct((_M, _D), jnp.float32),
    )(xm, mon, vec(time_emb_w), vec(time_emb_b), vec(value_emb_w),
      vec(value_emb_b), vec(empty_token), vec(unmonitored_token))
    return out.reshape(_N, _T, _P, _D)


# speedup vs baseline: 1.2344x; 1.2184x over previous
import jax
import jax.numpy as jnp
from jax.experimental import pallas as pl

_N, _T, _P, _D = 8, 48, 325, 512
_M = _N * _T * _P

_BM = 1920
_GRID = _M // _BM
_K = 16


def _split(a):
    hi = a.astype(jnp.bfloat16).astype(jnp.float32)
    return hi, a - hi


def _body(x_ref, mon_ref, b_ref, out_ref):
    v = x_ref[0:1, :]
    t = x_ref[1:2, :]
    monb = mon_ref[0:1, :] > 0.5
    inv = jnp.isnan(v)
    coef = jnp.where(monb & ~inv, v, 0.0)
    ones = jnp.ones_like(t)
    m_emp = jnp.where(monb & inv, 1.0, 0.0)
    m_un = jnp.where(monb, 0.0, 1.0)
    zero = jnp.zeros_like(t)

    t_hi, t_lo = _split(t)
    c_hi, c_lo = _split(coef)
    at = jnp.concatenate(
        [t_hi, t_hi, t_lo, c_hi, c_hi, c_lo, ones, ones,
         m_emp, m_emp, m_un, m_un, zero, zero, zero, zero],
        axis=0).astype(jnp.bfloat16)
    out_ref[...] = jax.lax.dot_general(
        at, b_ref[...], (((0,), (0,)), ((), ())),
        preferred_element_type=jnp.float32)


def kernel(x, monitor_mask, time_emb_w, time_emb_b, value_emb_w, value_emb_b,
           empty_token, unmonitored_token):
    xt = x.reshape(_M, 2).T
    mon = monitor_mask.reshape(1, _M).astype(jnp.float32)

    tw = time_emb_w.reshape(_D)
    vw = value_emb_w.reshape(_D)
    bias0 = (time_emb_b + value_emb_b).reshape(_D)
    bias_e = (empty_token - value_emb_b.reshape(_D))
    bias_u = (unmonitored_token - value_emb_b.reshape(_D))
    zero = jnp.zeros(_D, jnp.float32)

    def hi(a):
        return a.astype(jnp.bfloat16).astype(jnp.float32)

    rows = [hi(tw), tw - hi(tw), hi(tw),
            hi(vw), vw - hi(vw), hi(vw),
            hi(bias0), bias0 - hi(bias0),
            hi(bias_e), bias_e - hi(bias_e),
            hi(bias_u), bias_u - hi(bias_u),
            zero, zero, zero, zero]
    bmat = jnp.stack(rows, axis=0).astype(jnp.bfloat16)

    out = pl.pallas_call(
        _body,
        grid=(_GRID,),
        in_specs=[pl.BlockSpec((2, _BM), lambda i: (0, i)),
                  pl.BlockSpec((1, _BM), lambda i: (0, i)),
                  pl.BlockSpec((_K, _D), lambda i: (0, 0))],
        out_specs=pl.BlockSpec((_BM, _D), lambda i: (i, 0)),
        out_shape=jax.ShapeDtypeStruct((_M, _D), jnp.float32),
    )(xt, mon, bmat)
    return out.reshape(_N, _T, _P, _D)
